# Initial kernel scaffold; baseline (speedup 1.0000x reference)
import jax
import jax.numpy as jnp
from jax.experimental import pallas as pl


def kernel(x_res, x_AA, params, edge_index_contact, edge_index_backbone):
    raise NotImplementedError("write your pallas kernel here")


# speedup vs baseline: 3.8974x; 3.8974x over previous
import functools

import jax
import jax.numpy as jnp
from jax import lax
from jax.experimental import pallas as pl
from jax.experimental.pallas import tpu as pltpu
from jax.experimental.pallas import tpu_sc as plsc

N = 10000
E = 160000
D = 256
HD = 128
EH = 100
AAD = 20
OUTD = 32
KC = 64
CC = 0.25
EPS = 1e-5

NC = 2
NS = 16
CH = 80
EPT = E // NS
NCH = EPT // CH
RSTRIDE = 624
RSPAN = 640

BLK = 1000
NBLK = N // BLK

_SQRT_HALF = 0.7071067811865476


def _gelu(x):
    return 0.5 * x * (1.0 - lax.erf(-x * _SQRT_HALF))


def _ln_rows(x, g, b):
    m = jnp.mean(x, axis=-1, keepdims=True)
    v = jnp.mean((x - m) ** 2, axis=-1, keepdims=True)
    return (x - m) / jnp.sqrt(v + EPS) * g + b


def _dot(a, b):
    return jnp.dot(a, b, preferred_element_type=jnp.float32)


def _stem_body(x_ref, lng, lnb, ilng, ilnb, w1, b1, w2, b2, ha_ref, hb_ref):
    x = _ln_rows(x_ref[...], lng[...], lnb[...])
    h = _ln_rows(x, ilng[...], ilnb[...])
    h = _gelu(_dot(h, w1[...]) + b1[...])
    h = _gelu(_dot(h, w2[...]) + b2[...])
    ha_ref[...] = h[:, :HD]
    hb_ref[...] = h[:, HD:]


def _fixed(shape):
    return pl.BlockSpec(shape, lambda i: (0,) * len(shape))


def _rows(w):
    return pl.BlockSpec((BLK, w), lambda i: (i, 0))


def _stem(x_res, p):
    vec = lambda a: a.reshape(1, -1)
    return pl.pallas_call(
        _stem_body,
        grid=(NBLK,),
        in_specs=[
            _rows(D),
            _fixed((1, D)), _fixed((1, D)), _fixed((1, D)), _fixed((1, D)),
            _fixed((D, 2 * D)), _fixed((1, 2 * D)),
            _fixed((2 * D, D)), _fixed((1, D)),
        ],
        out_specs=[_rows(HD), _rows(HD)],
        out_shape=[
            jax.ShapeDtypeStruct((N, HD), jnp.float32),
            jax.ShapeDtypeStruct((N, HD), jnp.float32),
        ],
    )(x_res, vec(p['ln_g']), vec(p['ln_b']), vec(p['in_ln_g']),
      vec(p['in_ln_b']), p['in_w1'], vec(p['in_b1']), p['in_w2'],
      vec(p['in_b2']))


def _l1_body(ha, hb, ac0, ac1, ab0, ab1, cc_ref, cb_ref,
             wlc, wlb, wrc, wrb, blc, blb, g_ref):
    h = jnp.concatenate([ha[...], hb[...]], axis=1)
    agg_c = jnp.concatenate([ac0[...], ac1[...]], axis=1)
    agg_b = jnp.concatenate([ab0[...], ab1[...]], axis=1)
    aggm_c = agg_c / jnp.maximum(cc_ref[...][:, 0:1], 1.0)
    aggm_b = agg_b / jnp.maximum(cb_ref[...][:, 0:1], 1.0)
    out_c = _dot(aggm_c, wlc[...]) + blc[...] + _dot(h, wrc[...])
    out_b = _dot(aggm_b, wlb[...]) + blb[...] + _dot(h, wrb[...])
    g_ref[...] = _gelu((out_c + out_b) / 2.0)


def _l1(layer, p, ha, hb, ac0, ac1, ab0, ab1, cnt_c, cnt_b):
    vec = lambda a: a.reshape(1, -1)
    return pl.pallas_call(
        _l1_body,
        grid=(NBLK,),
        in_specs=[
            _rows(HD), _rows(HD), _rows(HD), _rows(HD), _rows(HD), _rows(HD),
            _rows(16), _rows(16),
            _fixed((D, D)), _fixed((D, D)), _fixed((D, D)), _fixed((D, D)),
            _fixed((1, D)), _fixed((1, D)),
        ],
        out_specs=[_rows(D)],
        out_shape=[jax.ShapeDtypeStruct((N, D), jnp.float32)],
    )(ha, hb, ac0, ac1, ab0, ab1, cnt_c, cnt_b,
      p['conv%d_contact_wl' % layer], p['conv%d_backbone_wl' % layer],
      p['conv%d_contact_wr' % layer], p['conv%d_backbone_wr' % layer],
      vec(p['conv%d_contact_bl' % layer]),
      vec(p['conv%d_backbone_bl' % layer]))[0]


def _gstats_body(g_ref, ms_ref, stats_ref):
    g = g_ref[...]
    mean = jnp.mean(g, axis=0, keepdims=True)
    out = g - mean * ms_ref[...]
    var = jnp.mean(out ** 2, axis=0, keepdims=True)
    stats_ref[...] = jnp.concatenate([mean, var] * 4, axis=0)


def _gstats(layer, p, g):
    vec = lambda a: a.reshape(1, -1)
    return pl.pallas_call(
        _gstats_body,
        out_shape=jax.ShapeDtypeStruct((8, D), jnp.float32),
    )(g, vec(p['gn%d_ms' % layer]))


def _l2_body(g_ref, stats_ref, w, b, ms_ref, ha_ref, hb_ref):
    g = g_ref[...]
    mean = stats_ref[...][0:1, :]
    var = stats_ref[...][1:2, :]
    out = g - mean * ms_ref[...]
    h = w[...] * out / jnp.sqrt(var + EPS) + b[...]
    ha_ref[...] = h[:, :HD]
    hb_ref[...] = h[:, HD:]


def _l2(layer, p, g, stats):
    vec = lambda a: a.reshape(1, -1)
    return pl.pallas_call(
        _l2_body,
        grid=(NBLK,),
        in_specs=[
            _rows(D),
            _fixed((8, D)),
            _fixed((1, D)), _fixed((1, D)), _fixed((1, D)),
        ],
        out_specs=[_rows(HD), _rows(HD)],
        out_shape=[
            jax.ShapeDtypeStruct((N, HD), jnp.float32),
            jax.ShapeDtypeStruct((N, HD), jnp.float32),
        ],
    )(g, stats, vec(p['gn%d_w' % layer]), vec(p['gn%d_b' % layer]),
      vec(p['gn%d_ms' % layer]))


def _head_body(h1a, h1b, h2a, h2b, aa,
               lw1, lb1, lw2, lb2,
               ow1, ob1, ow2, ob2, ow3, ob3,
               cb, cbt, q_ref, loss_ref):
    i = pl.program_id(0)
    x = jnp.concatenate([h1a[...], h1b[...], h2a[...], h2b[...]], axis=1)
    x = _gelu(_dot(x, lw1[...]) + lb1[...])
    x = _gelu(_dot(x, lw2[...]) + lb2[...])
    x = jnp.concatenate([x, aa[...]], axis=1)
    x = _gelu(_dot(x, ow1[...]) + ob1[...])
    x = _gelu(_dot(x, ow2[...]) + ob2[...])
    z = jnp.tanh(_dot(x, ow3[...]) + ob3[...])
    cbsq = jnp.sum(cbt[...] * cbt[...], axis=0, keepdims=True)
    d = (jnp.sum(z * z, axis=1, keepdims=True) + cbsq - 2.0 * _dot(z, cbt[...]))
    dmin = jnp.min(d, axis=1, keepdims=True)
    iota = lax.broadcasted_iota(jnp.int32, (BLK, KC), 1)
    idx = jnp.min(jnp.where(d == dmin, iota, KC), axis=1, keepdims=True)
    onehot = (iota == idx).astype(jnp.float32)
    q = jnp.dot(onehot, cb[...], preferred_element_type=jnp.float32,
                precision=lax.Precision.HIGHEST)
    q_ref[...] = z + (q - z)
    sq = jnp.sum((q - z) ** 2)

    @pl.when(i == 0)
    def _():
        loss_ref[...] = jnp.zeros_like(loss_ref)

    loss_ref[...] += jnp.full((8, 128), sq, jnp.float32)

    @pl.when(i == NBLK - 1)
    def _():
        m = loss_ref[...] * (1.0 / (N * OUTD))
        loss_ref[...] = m + CC * m


def _head(p, saves, x_AA):
    vec = lambda a: a.reshape(1, -1)
    (h1a, h1b), (h2a, h2b) = saves
    cb = p['codebook']
    return pl.pallas_call(
        _head_body,
        grid=(NBLK,),
        in_specs=[
            _rows(HD), _rows(HD), _rows(HD), _rows(HD), _rows(AAD),
            _fixed((2 * D, EH)), _fixed((1, EH)),
            _fixed((EH, EH)), _fixed((1, EH)),
            _fixed((EH + AAD, EH)), _fixed((1, EH)),
            _fixed((EH, EH)), _fixed((1, EH)),
            _fixed((EH, OUTD)), _fixed((1, OUTD)),
            _fixed((KC, OUTD)), _fixed((OUTD, KC)),
        ],
        out_specs=[
            _rows(OUTD),
            pl.BlockSpec((8, 128), lambda i: (0, 0)),
        ],
        out_shape=[
            jax.ShapeDtypeStruct((N, OUTD), jnp.float32),
            jax.ShapeDtypeStruct((8, 128), jnp.float32),
        ],
    )(h1a, h1b, h2a, h2b, x_AA,
      p['lin_w1'], vec(p['lin_b1']), p['lin_w2'], vec(p['lin_b2']),
      p['od_w1'], vec(p['od_b1']), p['od_w2'], vec(p['od_b2']),
      p['od_w3'], vec(p['od_b3']),
      cb, cb.T)


def _counts_body(dstc_hbm, dstb_hbm, outc_hbm, outb_hbm, idx_v, ones_v, acc,
                 sem):
    c = lax.axis_index("c")
    s = lax.axis_index("s")

    def fill(val):
        def body(i, _):
            ones_v[i, :] = jnp.full((16,), val, jnp.float32)
            return 0
        lax.fori_loop(0, CH, body, 0)

    fill(0.0)
    base = pl.multiple_of(s * RSTRIDE, 8)
    for t in range(RSPAN // CH):
        pltpu.sync_copy(ones_v, acc.at[pl.ds(base + t * CH, CH)])
    fill(1.0)

    @pl.when(c == 0)
    def _():
        pltpu.sync_copy(dstc_hbm.at[s], idx_v)

    @pl.when(c == 1)
    def _():
        pltpu.sync_copy(dstb_hbm.at[s], idx_v)

    plsc.subcore_barrier()

    def chunk(k, _):
        pltpu.sync_copy(ones_v, acc.at[idx_v.at[k]], add=True)
        return 0

    lax.fori_loop(0, NCH, chunk, 0)
    plsc.subcore_barrier()

    @pl.when(c == 0)
    def _():
        pltpu.sync_copy(acc.at[pl.ds(base, RSPAN)],
                        outc_hbm.at[pl.ds(base, RSPAN)])

    @pl.when(c == 1)
    def _():
        pltpu.sync_copy(acc.at[pl.ds(base, RSPAN)],
                        outb_hbm.at[pl.ds(base, RSPAN)])


def _counts(dst_c3, dst_b3):
    mesh = plsc.VectorSubcoreMesh(core_axis_name="c", subcore_axis_name="s")
    kfn = pl.kernel(
        _counts_body,
        out_type=[
            jax.ShapeDtypeStruct((N, 16), jnp.float32),
            jax.ShapeDtypeStruct((N, 16), jnp.float32),
        ],
        mesh=mesh,
        scratch_types=[
            pltpu.VMEM((NCH, CH), jnp.int32),
            pltpu.VMEM((CH, 16), jnp.float32),
            pltpu.VMEM_SHARED((N, 16), jnp.float32),
            pltpu.SemaphoreType.DMA,
        ],
    )
    return kfn(dst_c3, dst_b3)


def _seg_body(ha_hbm, hb_hbm, src_hbm, dst_hbm, outa_hbm, outb_hbm,
              sidx_v, didx_v, rows_v, acc, sem):
    c = lax.axis_index("c")
    s = lax.axis_index("s")

    def fillz(i, _):
        for j in range(HD // 16):
            rows_v[i, pl.ds(j * 16, 16)] = jnp.zeros((16,), jnp.float32)
        return 0

    lax.fori_loop(0, CH, fillz, 0)
    base = pl.multiple_of(s * RSTRIDE, 8)
    for t in range(RSPAN // CH):
        pltpu.sync_copy(rows_v, acc.at[pl.ds(base + t * CH, CH)])

    pltpu.sync_copy(src_hbm.at[s], sidx_v)
    pltpu.sync_copy(dst_hbm.at[s], didx_v)
    plsc.subcore_barrier()

    def chunk(k, _):
        @pl.when(c == 0)
        def _():
            pltpu.async_copy(ha_hbm.at[sidx_v.at[k]], rows_v, sem).wait()

        @pl.when(c == 1)
        def _():
            pltpu.async_copy(hb_hbm.at[sidx_v.at[k]], rows_v, sem).wait()

        pltpu.sync_copy(rows_v, acc.at[didx_v.at[k]], add=True)
        return 0

    lax.fori_loop(0, NCH, chunk, 0)
    plsc.subcore_barrier()

    @pl.when(c == 0)
    def _():
        pltpu.sync_copy(acc.at[pl.ds(base, RSPAN)],
                        outa_hbm.at[pl.ds(base, RSPAN)])

    @pl.when(c == 1)
    def _():
        pltpu.sync_copy(acc.at[pl.ds(base, RSPAN)],
                        outb_hbm.at[pl.ds(base, RSPAN)])


def _segsum(ha, hb, src3, dst3):
    mesh = plsc.VectorSubcoreMesh(core_axis_name="c", subcore_axis_name="s")
    kfn = pl.kernel(
        _seg_body,
        out_type=[
            jax.ShapeDtypeStruct((N, HD), jnp.float32),
            jax.ShapeDtypeStruct((N, HD), jnp.float32),
        ],
        mesh=mesh,
        scratch_types=[
            pltpu.VMEM((NCH, CH), jnp.int32),
            pltpu.VMEM((NCH, CH), jnp.int32),
            pltpu.VMEM((CH, HD), jnp.float32),
            pltpu.VMEM_SHARED((N, HD), jnp.float32),
            pltpu.SemaphoreType.DMA,
        ],
    )
    return kfn(ha, hb, src3, dst3)


def kernel(x_res, x_AA, params, edge_index_contact, edge_index_backbone):
    p = params
    src_c = edge_index_contact[0].reshape(NS, NCH, CH)
    dst_c = edge_index_contact[1].reshape(NS, NCH, CH)
    src_b = edge_index_backbone[0].reshape(NS, NCH, CH)
    dst_b = edge_index_backbone[1].reshape(NS, NCH, CH)

    ha, hb = _stem(x_res, p)
    cnt_c, cnt_b = _counts(dst_c, dst_b)

    saves = []
    for layer in range(2):
        ac0, ac1 = _segsum(ha, hb, src_c, dst_c)
        ab0, ab1 = _segsum(ha, hb, src_b, dst_b)
        g = _l1(layer, p, ha, hb, ac0, ac1, ab0, ab1, cnt_c, cnt_b)
        stats = _gstats(layer, p, g)
        ha, hb = _l2(layer, p, g, stats)
        saves.append((ha, hb))

    qst, lossbuf = _head(p, saves, x_AA)
    return qst, lossbuf[0, 0]
